# Initial kernel scaffold; baseline (speedup 1.0000x reference)
import jax
import jax.numpy as jnp
from jax.experimental import pallas as pl


def kernel(x, edge_index, W, attn_l, attn_r, bias):
    raise NotImplementedError("write your pallas kernel here")


# speedup vs baseline: 19.3293x; 19.3293x over previous
import functools

import jax
import jax.numpy as jnp
from jax import lax
from jax.experimental import pallas as pl
from jax.experimental.pallas import tpu as pltpu
from jax.experimental.pallas import tpu_sc as plsc

N = 10000
E = 320000
D = 128
NEG_SLOPE = 0.2

NC = 2
NS = 16
L = 16
NW = NC * NS
EPW = E // NW
K = 80
NCH = EPW // K
ZB = 40
NZC = N // ZB

BR = 1000

_DN = jax.lax.GatherDimensionNumbers(offset_dims=(), collapsed_slice_dims=(0,),
                                     start_index_map=(0,))


def _prep_body(x_ref, w_ref, al_ref, ar_ref, feat_ref, el_ref, er_ref,
               ml_ref, mr_ref):
    i = pl.program_id(0)
    feat = jnp.dot(x_ref[...], w_ref[...], preferred_element_type=jnp.float32)
    feat_ref[...] = feat
    el = jnp.dot(feat, al_ref[...], preferred_element_type=jnp.float32)
    er = jnp.dot(feat, ar_ref[...], preferred_element_type=jnp.float32)
    el_ref[...] = el
    er_ref[...] = er

    @pl.when(i == 0)
    def _():
        ml_ref[0, 0] = jnp.float32(-jnp.inf)
        mr_ref[0, 0] = jnp.float32(-jnp.inf)

    ml_ref[0, 0] = jnp.maximum(ml_ref[0, 0], jnp.max(el))
    mr_ref[0, 0] = jnp.maximum(mr_ref[0, 0], jnp.max(er))


@jax.jit
def _prep(x, W, al, ar):
    return pl.pallas_call(
        _prep_body,
        grid=(N // BR,),
        in_specs=[
            pl.BlockSpec((BR, D), lambda i: (i, 0)),
            pl.BlockSpec((D, D), lambda i: (0, 0)),
            pl.BlockSpec((D, 1), lambda i: (0, 0)),
            pl.BlockSpec((D, 1), lambda i: (0, 0)),
        ],
        out_specs=[
            pl.BlockSpec((BR, D), lambda i: (i, 0)),
            pl.BlockSpec((BR, 1), lambda i: (i, 0)),
            pl.BlockSpec((BR, 1), lambda i: (i, 0)),
            pl.BlockSpec(memory_space=pltpu.SMEM),
            pl.BlockSpec(memory_space=pltpu.SMEM),
        ],
        out_shape=[
            jax.ShapeDtypeStruct((N, D), jnp.float32),
            jax.ShapeDtypeStruct((N, 1), jnp.float32),
            jax.ShapeDtypeStruct((N, 1), jnp.float32),
            jax.ShapeDtypeStruct((1, 1), jnp.float32),
            jax.ShapeDtypeStruct((1, 1), jnp.float32),
        ],
    )(x, W, al, ar)


def _sc_body(feat_h, el_h, er_h, src_h, dst_h, c_h, acc_out, asum_out,
             el_v, er_v, cbuf, srcb, dstb, rows, sbuf, abuf, zbuf, acc_sh,
             asum_sh, gsem):
    cid = lax.axis_index("c")
    sid = lax.axis_index("s")
    wid = cid * NS + sid

    pltpu.sync_copy(el_h, el_v)
    pltpu.sync_copy(er_h, er_v)
    pltpu.sync_copy(c_h, cbuf)

    zv = jnp.zeros((L,), jnp.float32)

    def zb_body(i, carry):
        for c in range(D // L):
            zbuf[i, pl.ds(c * L, L)] = zv
        return carry

    lax.fori_loop(0, ZB, zb_body, 0)

    def zc_body(j, carry):
        @pl.when(j % NS == sid)
        def _():
            pltpu.sync_copy(zbuf, acc_sh.at[pl.ds(j * ZB, ZB)])
            pltpu.sync_copy(zbuf.at[0, pl.ds(0, ZB)],
                            asum_sh.at[pl.ds(j * ZB, ZB)])
        return carry

    lax.fori_loop(0, NZC, zc_body, 0)
    plsc.subcore_barrier()

    cv = cbuf[...]
    lane = lax.iota(jnp.int32, L)
    ebase = wid * EPW

    def chunk(t, carry):
        base = ebase + t * K
        pltpu.sync_copy(src_h.at[pl.ds(base, K)], srcb)
        pltpu.sync_copy(dst_h.at[pl.ds(base, K)], dstb)
        cp = pltpu.async_copy(feat_h.at[srcb], rows, gsem)
        avs = []
        for g in range(K // L):
            sidx = srcb[pl.ds(g * L, L)]
            didx = dstb[pl.ds(g * L, L)]
            e = plsc.load_gather(el_v, [sidx]) + plsc.load_gather(er_v, [didx])
            e = jnp.where(e > 0, e, NEG_SLOPE * e)
            a = jnp.exp(e - cv)
            abuf[pl.ds(g * L, L)] = a
            avs.append(a)
        cp.wait()
        for g in range(K // L):
            ag = avs[g]
            for j in range(L):
                i = g * L + j
                av = lax.gather(ag, jnp.full((L, 1), j, jnp.int32), _DN, (1,),
                                mode=lax.GatherScatterMode.PROMISE_IN_BOUNDS)
                for c in range(D // L):
                    sbuf[i, pl.ds(c * L, L)] = rows[i, pl.ds(c * L, L)] * av
        pltpu.sync_copy(sbuf, acc_sh.at[dstb], add=True)
        pltpu.sync_copy(abuf, asum_sh.at[dstb], add=True)
        return carry

    lax.fori_loop(0, NCH, chunk, 0)
    plsc.subcore_barrier()

    def out_body(j, carry):
        @pl.when(j % NS == sid)
        def _():
            pltpu.sync_copy(acc_sh.at[pl.ds(j * ZB, ZB)],
                            acc_out.at[cid, pl.ds(j * ZB, ZB)])
            ab = abuf.at[pl.ds(0, ZB)]
            pltpu.sync_copy(asum_sh.at[pl.ds(j * ZB, ZB)], ab)
            pltpu.sync_copy(ab,
                            asum_out.at[pl.ds(cid * N + j * ZB, ZB)])
        return carry

    lax.fori_loop(0, NZC, out_body, 0)


@jax.jit
def _sc_edges(feat, el, er, src, dst, cvec):
    mesh = plsc.VectorSubcoreMesh(core_axis_name="c", subcore_axis_name="s")
    return pl.kernel(
        _sc_body,
        out_type=(jax.ShapeDtypeStruct((NC, N, D), jnp.float32),
                  jax.ShapeDtypeStruct((NC * N,), jnp.float32)),
        mesh=mesh,
        scratch_types=[
            pltpu.VMEM((N,), jnp.float32),
            pltpu.VMEM((N,), jnp.float32),
            pltpu.VMEM((L,), jnp.float32),
            pltpu.VMEM((K,), jnp.int32),
            pltpu.VMEM((K,), jnp.int32),
            pltpu.VMEM((K, D), jnp.float32),
            pltpu.VMEM((K, D), jnp.float32),
            pltpu.VMEM((K,), jnp.float32),
            pltpu.VMEM((ZB, D), jnp.float32),
            pltpu.VMEM_SHARED((N, D), jnp.float32),
            pltpu.VMEM_SHARED((N,), jnp.float32),
            pltpu.SemaphoreType.DMA,
        ],
        compiler_params=pltpu.CompilerParams(needs_layout_passes=False),
    )(feat, el, er, src, dst, cvec)


def _finish_body(acc_ref, asum_ref, x_ref, b_ref, o_ref):
    s = acc_ref[0] + acc_ref[1]
    denom = asum_ref[0] + asum_ref[1] + jnp.float32(1e-9)
    o_ref[...] = s / denom + x_ref[...] + b_ref[...]


@jax.jit
def _finish(acc, asum, x, bias2):
    return pl.pallas_call(
        _finish_body,
        grid=(N // BR,),
        in_specs=[
            pl.BlockSpec((NC, BR, D), lambda i: (0, i, 0)),
            pl.BlockSpec((NC, BR, 1), lambda i: (0, i, 0)),
            pl.BlockSpec((BR, D), lambda i: (i, 0)),
            pl.BlockSpec((1, D), lambda i: (0, 0)),
        ],
        out_specs=pl.BlockSpec((BR, D), lambda i: (i, 0)),
        out_shape=jax.ShapeDtypeStruct((N, D), jnp.float32),
    )(acc, asum, x, bias2)


def kernel(x, edge_index, W, attn_l, attn_r, bias):
    src = edge_index[0].astype(jnp.int32)
    dst = edge_index[1].astype(jnp.int32)
    feat, el2, er2, ml, mr = _prep(x, W, attn_l.reshape(D, 1),
                                   attn_r.reshape(D, 1))
    C = jnp.maximum(ml[0, 0] + mr[0, 0], jnp.float32(0.0))
    cvec = jnp.full((L,), C, jnp.float32)
    acc, asum = _sc_edges(feat, el2.reshape(N), er2.reshape(N), src, dst,
                          cvec)
    return _finish(acc, asum.reshape(NC, N, 1), x, bias.reshape(1, D))


# speedup vs baseline: 24.8831x; 1.2873x over previous
import functools

import jax
import jax.numpy as jnp
from jax import lax
from jax.experimental import pallas as pl
from jax.experimental.pallas import tpu as pltpu
from jax.experimental.pallas import tpu_sc as plsc

N = 10000
E = 320000
D = 128
NEG_SLOPE = 0.2

NC = 2
NS = 16
L = 16
NW = NC * NS
EPW = E // NW
K = 80
NCH = EPW // K
ZB = 40
NZC = N // ZB

BR = 1000

_DN = jax.lax.GatherDimensionNumbers(offset_dims=(), collapsed_slice_dims=(0,),
                                     start_index_map=(0,))


def _prep_body(x_ref, w_ref, al_ref, ar_ref, feat_ref, el_ref, er_ref,
               ml_ref, mr_ref):
    i = pl.program_id(0)
    feat = jnp.dot(x_ref[...], w_ref[...], preferred_element_type=jnp.float32)
    feat_ref[...] = feat
    el = jnp.dot(feat, al_ref[...], preferred_element_type=jnp.float32)
    er = jnp.dot(feat, ar_ref[...], preferred_element_type=jnp.float32)
    el_ref[...] = el
    er_ref[...] = er

    @pl.when(i == 0)
    def _():
        ml_ref[0, 0] = jnp.float32(-jnp.inf)
        mr_ref[0, 0] = jnp.float32(-jnp.inf)

    ml_ref[0, 0] = jnp.maximum(ml_ref[0, 0], jnp.max(el))
    mr_ref[0, 0] = jnp.maximum(mr_ref[0, 0], jnp.max(er))


@jax.jit
def _prep(x, W, al, ar):
    return pl.pallas_call(
        _prep_body,
        grid=(N // BR,),
        in_specs=[
            pl.BlockSpec((BR, D), lambda i: (i, 0)),
            pl.BlockSpec((D, D), lambda i: (0, 0)),
            pl.BlockSpec((D, 1), lambda i: (0, 0)),
            pl.BlockSpec((D, 1), lambda i: (0, 0)),
        ],
        out_specs=[
            pl.BlockSpec((BR, D), lambda i: (i, 0)),
            pl.BlockSpec((BR, 1), lambda i: (i, 0)),
            pl.BlockSpec((BR, 1), lambda i: (i, 0)),
            pl.BlockSpec(memory_space=pltpu.SMEM),
            pl.BlockSpec(memory_space=pltpu.SMEM),
        ],
        out_shape=[
            jax.ShapeDtypeStruct((N, D), jnp.float32),
            jax.ShapeDtypeStruct((N, 1), jnp.float32),
            jax.ShapeDtypeStruct((N, 1), jnp.float32),
            jax.ShapeDtypeStruct((1, 1), jnp.float32),
            jax.ShapeDtypeStruct((1, 1), jnp.float32),
        ],
    )(x, W, al, ar)


def _sc_body(feat_h, el_h, er_h, src_h, dst_h, c_h, acc_out, asum_out,
             el_v, er_v, cbuf, srcb0, srcb1, dstb0, dstb1, rows0, rows1,
             abuf, zbuf, acc_sh, asum_sh, gsem0, gsem1, isem0, isem1):
    cid = lax.axis_index("c")
    sid = lax.axis_index("s")
    wid = cid * NS + sid

    pltpu.sync_copy(el_h, el_v)
    pltpu.sync_copy(er_h, er_v)
    pltpu.sync_copy(c_h, cbuf)

    zv = jnp.zeros((L,), jnp.float32)

    def zb_body(i, carry):
        for c in range(D // L):
            zbuf[i, pl.ds(c * L, L)] = zv
        return carry

    lax.fori_loop(0, ZB, zb_body, 0)

    def zc_body(j, carry):
        @pl.when(j % NS == sid)
        def _():
            pltpu.sync_copy(zbuf, acc_sh.at[pl.ds(j * ZB, ZB)])
            pltpu.sync_copy(zbuf.at[0, pl.ds(0, ZB)],
                            asum_sh.at[pl.ds(j * ZB, ZB)])
        return carry

    lax.fori_loop(0, NZC, zc_body, 0)
    plsc.subcore_barrier()

    cv = cbuf[...]
    ebase = wid * EPW

    srcb = [srcb0, srcb1]
    dstb = [dstb0, dstb1]
    rows = [rows0, rows1]
    gsem = [gsem0, gsem1]
    isem = [isem0, isem1]

    def idx_load(c, par, sync):
        base = ebase + c * K
        if sync:
            pltpu.sync_copy(src_h.at[pl.ds(base, K)], srcb[par])
            pltpu.sync_copy(dst_h.at[pl.ds(base, K)], dstb[par])
        else:
            pltpu.async_copy(src_h.at[pl.ds(base, K)], srcb[par], isem[par])
            pltpu.async_copy(dst_h.at[pl.ds(base, K)], dstb[par], isem[par])

    def idx_drain(par):
        pltpu.make_async_copy(src_h.at[pl.ds(0, K)], srcb[par],
                              isem[par]).wait()
        pltpu.make_async_copy(dst_h.at[pl.ds(0, K)], dstb[par],
                              isem[par]).wait()

    def gather_start(c, par):
        pltpu.async_copy(feat_h.at[srcb[par]], rows[par], gsem[par])

    def phase(c, par):
        nxt = 1 - par

        @pl.when(c < NCH)
        def _():
            @pl.when(c + 1 < NCH)
            def _():
                idx_drain(nxt)
                gather_start(c + 1, nxt)

            avs = []
            for g in range(K // L):
                sidx = srcb[par][pl.ds(g * L, L)]
                didx = dstb[par][pl.ds(g * L, L)]
                e = (plsc.load_gather(el_v, [sidx]) +
                     plsc.load_gather(er_v, [didx]))
                e = jnp.where(e > 0, e, NEG_SLOPE * e)
                a = jnp.exp(e - cv)
                abuf[pl.ds(g * L, L)] = a
                avs.append(a)

            pltpu.make_async_copy(feat_h.at[srcb[par]], rows[par],
                                  gsem[par]).wait()

            r = rows[par]
            for g in range(K // L):
                ag = avs[g]
                for j in range(L):
                    i = g * L + j
                    av = lax.gather(ag, jnp.full((L, 1), j, jnp.int32), _DN,
                                    (1,),
                                    mode=lax.GatherScatterMode.PROMISE_IN_BOUNDS)
                    for cc in range(D // L):
                        r[i, pl.ds(cc * L, L)] = r[i, pl.ds(cc * L, L)] * av

            pltpu.sync_copy(r, acc_sh.at[dstb[par]], add=True)
            pltpu.sync_copy(abuf, asum_sh.at[dstb[par]], add=True)

            @pl.when(c + 2 < NCH)
            def _():
                idx_load(c + 2, par, sync=False)

    idx_load(0, 0, sync=True)
    gather_start(0, 0)
    if NCH > 1:
        idx_load(1, 1, sync=False)

    def pair(p, carry):
        phase(2 * p, 0)
        phase(2 * p + 1, 1)
        return carry

    lax.fori_loop(0, (NCH + 1) // 2, pair, 0)
    plsc.subcore_barrier()

    def out_body(j, carry):
        @pl.when(j % NS == sid)
        def _():
            pltpu.sync_copy(acc_sh.at[pl.ds(j * ZB, ZB)],
                            acc_out.at[cid, pl.ds(j * ZB, ZB)])
            ab = abuf.at[pl.ds(0, ZB)]
            pltpu.sync_copy(asum_sh.at[pl.ds(j * ZB, ZB)], ab)
            pltpu.sync_copy(ab,
                            asum_out.at[pl.ds(cid * N + j * ZB, ZB)])
        return carry

    lax.fori_loop(0, NZC, out_body, 0)


@jax.jit
def _sc_edges(feat, el, er, src, dst, cvec):
    mesh = plsc.VectorSubcoreMesh(core_axis_name="c", subcore_axis_name="s")
    return pl.kernel(
        _sc_body,
        out_type=(jax.ShapeDtypeStruct((NC, N, D), jnp.float32),
                  jax.ShapeDtypeStruct((NC * N,), jnp.float32)),
        mesh=mesh,
        scratch_types=[
            pltpu.VMEM((N,), jnp.float32),
            pltpu.VMEM((N,), jnp.float32),
            pltpu.VMEM((L,), jnp.float32),
            pltpu.VMEM((K,), jnp.int32),
            pltpu.VMEM((K,), jnp.int32),
            pltpu.VMEM((K,), jnp.int32),
            pltpu.VMEM((K,), jnp.int32),
            pltpu.VMEM((K, D), jnp.float32),
            pltpu.VMEM((K, D), jnp.float32),
            pltpu.VMEM((K,), jnp.float32),
            pltpu.VMEM((ZB, D), jnp.float32),
            pltpu.VMEM_SHARED((N, D), jnp.float32),
            pltpu.VMEM_SHARED((N,), jnp.float32),
            pltpu.SemaphoreType.DMA,
            pltpu.SemaphoreType.DMA,
            pltpu.SemaphoreType.DMA,
            pltpu.SemaphoreType.DMA,
        ],
        compiler_params=pltpu.CompilerParams(needs_layout_passes=False),
    )(feat, el, er, src, dst, cvec)


def _finish_body(acc_ref, asum_ref, x_ref, b_ref, o_ref):
    s = acc_ref[0] + acc_ref[1]
    denom = asum_ref[0] + asum_ref[1] + jnp.float32(1e-9)
    o_ref[...] = s / denom + x_ref[...] + b_ref[...]


@jax.jit
def _finish(acc, asum, x, bias2):
    return pl.pallas_call(
        _finish_body,
        grid=(N // BR,),
        in_specs=[
            pl.BlockSpec((NC, BR, D), lambda i: (0, i, 0)),
            pl.BlockSpec((NC, BR, 1), lambda i: (0, i, 0)),
            pl.BlockSpec((BR, D), lambda i: (i, 0)),
            pl.BlockSpec((1, D), lambda i: (0, 0)),
        ],
        out_specs=pl.BlockSpec((BR, D), lambda i: (i, 0)),
        out_shape=jax.ShapeDtypeStruct((N, D), jnp.float32),
    )(acc, asum, x, bias2)


def kernel(x, edge_index, W, attn_l, attn_r, bias):
    src = edge_index[0].astype(jnp.int32)
    dst = edge_index[1].astype(jnp.int32)
    feat, el2, er2, ml, mr = _prep(x, W, attn_l.reshape(D, 1),
                                   attn_r.reshape(D, 1))
    C = jnp.maximum(ml[0, 0] + mr[0, 0], jnp.float32(0.0))
    cvec = jnp.full((L,), C, jnp.float32)
    acc, asum = _sc_edges(feat, el2.reshape(N), er2.reshape(N), src, dst,
                          cvec)
    return _finish(acc, asum.reshape(NC, N, 1), x, bias.reshape(1, D))


# speedup vs baseline: 29.3605x; 1.1799x over previous
import functools

import jax
import jax.numpy as jnp
from jax import lax
from jax.experimental import pallas as pl
from jax.experimental.pallas import tpu as pltpu
from jax.experimental.pallas import tpu_sc as plsc

N = 10000
E = 320000
D = 128
NEG_SLOPE = 0.2

NC = 2
NS = 16
L = 16
NW = NC * NS
EPW = E // NW
K = 80
NCH = EPW // K
ZB = 40
NZC = N // ZB

BR = 1000

_DN = jax.lax.GatherDimensionNumbers(offset_dims=(), collapsed_slice_dims=(0,),
                                     start_index_map=(0,))


def _prep_body(x_ref, w_ref, al_ref, ar_ref, feat_ref, el_ref, er_ref,
               ml_ref, mr_ref):
    i = pl.program_id(0)
    feat = jnp.dot(x_ref[...], w_ref[...], preferred_element_type=jnp.float32)
    feat_ref[...] = feat
    el = jnp.dot(feat, al_ref[...], preferred_element_type=jnp.float32)
    er = jnp.dot(feat, ar_ref[...], preferred_element_type=jnp.float32)
    el_ref[...] = el
    er_ref[...] = er

    @pl.when(i == 0)
    def _():
        ml_ref[0, 0] = jnp.float32(-jnp.inf)
        mr_ref[0, 0] = jnp.float32(-jnp.inf)

    ml_ref[0, 0] = jnp.maximum(ml_ref[0, 0], jnp.max(el))
    mr_ref[0, 0] = jnp.maximum(mr_ref[0, 0], jnp.max(er))


@jax.jit
def _prep(x, W, al, ar):
    return pl.pallas_call(
        _prep_body,
        grid=(N // BR,),
        in_specs=[
            pl.BlockSpec((BR, D), lambda i: (i, 0)),
            pl.BlockSpec((D, D), lambda i: (0, 0)),
            pl.BlockSpec((D, 1), lambda i: (0, 0)),
            pl.BlockSpec((D, 1), lambda i: (0, 0)),
        ],
        out_specs=[
            pl.BlockSpec((BR, D), lambda i: (i, 0)),
            pl.BlockSpec((BR, 1), lambda i: (i, 0)),
            pl.BlockSpec((BR, 1), lambda i: (i, 0)),
            pl.BlockSpec(memory_space=pltpu.SMEM),
            pl.BlockSpec(memory_space=pltpu.SMEM),
        ],
        out_shape=[
            jax.ShapeDtypeStruct((N, D), jnp.float32),
            jax.ShapeDtypeStruct((N, 1), jnp.float32),
            jax.ShapeDtypeStruct((N, 1), jnp.float32),
            jax.ShapeDtypeStruct((1, 1), jnp.float32),
            jax.ShapeDtypeStruct((1, 1), jnp.float32),
        ],
    )(x, W, al, ar)


def _sc_body(feat_h, el_h, er_h, src_h, dst_h, c_h, acc_out, asum_out,
             el_v, er_v, cbuf, srcb0, srcb1, dstb0, dstb1, dsts0, dsts1,
             rows0, rows1, abuf0, abuf1, zbuf, acc_sh, asum_sh,
             gsem0, gsem1, isem0, isem1, ssem0, ssem1, esem0, esem1):
    cid = lax.axis_index("c")
    sid = lax.axis_index("s")
    wid = cid * NS + sid

    pltpu.sync_copy(el_h, el_v)
    pltpu.sync_copy(er_h, er_v)
    pltpu.sync_copy(c_h, cbuf)

    zv = jnp.zeros((L,), jnp.float32)

    def zb_body(i, carry):
        for c in range(D // L):
            zbuf[i, pl.ds(c * L, L)] = zv
        return carry

    lax.fori_loop(0, ZB, zb_body, 0)

    def zc_body(j, carry):
        @pl.when(j % NS == sid)
        def _():
            pltpu.sync_copy(zbuf, acc_sh.at[pl.ds(j * ZB, ZB)])
            pltpu.sync_copy(zbuf.at[0, pl.ds(0, ZB)],
                            asum_sh.at[pl.ds(j * ZB, ZB)])
        return carry

    lax.fori_loop(0, NZC, zc_body, 0)
    plsc.subcore_barrier()

    cv = cbuf[...]
    ebase = wid * EPW

    srcb = [srcb0, srcb1]
    dstb = [dstb0, dstb1]
    dsts = [dsts0, dsts1]
    rows = [rows0, rows1]
    abuf = [abuf0, abuf1]
    gsem = [gsem0, gsem1]
    isem = [isem0, isem1]
    ssem = [ssem0, ssem1]
    esem = [esem0, esem1]

    def scat_drain(par):
        pltpu.make_async_copy(rows[par], acc_sh.at[dsts[par]],
                              ssem[par]).wait()
        pltpu.make_async_copy(abuf[par], asum_sh.at[dsts[par]],
                              esem[par]).wait()

    def idx_load(c, par, sync):
        base = ebase + c * K
        if sync:
            pltpu.sync_copy(src_h.at[pl.ds(base, K)], srcb[par])
            pltpu.sync_copy(dst_h.at[pl.ds(base, K)], dstb[par])
        else:
            pltpu.async_copy(src_h.at[pl.ds(base, K)], srcb[par], isem[par])
            pltpu.async_copy(dst_h.at[pl.ds(base, K)], dstb[par], isem[par])

    def idx_drain(par):
        pltpu.make_async_copy(src_h.at[pl.ds(0, K)], srcb[par],
                              isem[par]).wait()
        pltpu.make_async_copy(dst_h.at[pl.ds(0, K)], dstb[par],
                              isem[par]).wait()

    def gather_start(c, par):
        pltpu.async_copy(feat_h.at[srcb[par]], rows[par], gsem[par])

    def phase(c, par):
        nxt = 1 - par

        @pl.when(c < NCH)
        def _():
            @pl.when(c + 1 < NCH)
            def _():
                idx_drain(nxt)

            avs = []
            for g in range(K // L):
                sidx = srcb[par][pl.ds(g * L, L)]
                didx = dstb[par][pl.ds(g * L, L)]
                e = (plsc.load_gather(el_v, [sidx]) +
                     plsc.load_gather(er_v, [didx]))
                e = jnp.where(e > 0, e, NEG_SLOPE * e)
                a = jnp.exp(e - cv)
                abuf[par][pl.ds(g * L, L)] = a
                avs.append(a)

            @pl.when(c >= 1)
            def _():
                scat_drain(nxt)

            @pl.when(c + 1 < NCH)
            def _():
                gather_start(c + 1, nxt)

            pltpu.make_async_copy(feat_h.at[srcb[par]], rows[par],
                                  gsem[par]).wait()

            r = rows[par]
            for g in range(K // L):
                ag = avs[g]
                for j in range(L):
                    i = g * L + j
                    av = lax.gather(ag, jnp.full((L, 1), j, jnp.int32), _DN,
                                    (1,),
                                    mode=lax.GatherScatterMode.PROMISE_IN_BOUNDS)
                    for cc in range(D // L):
                        r[i, pl.ds(cc * L, L)] = r[i, pl.ds(cc * L, L)] * av

            for g in range(K // L):
                dsts[par][pl.ds(g * L, L)] = dstb[par][pl.ds(g * L, L)]
            pltpu.async_copy(r, acc_sh.at[dsts[par]], ssem[par], add=True)
            pltpu.async_copy(abuf[par], asum_sh.at[dsts[par]], esem[par],
                             add=True)

            @pl.when(c + 2 < NCH)
            def _():
                idx_load(c + 2, par, sync=False)

    idx_load(0, 0, sync=True)
    gather_start(0, 0)
    if NCH > 1:
        idx_load(1, 1, sync=False)

    def pair(p, carry):
        phase(2 * p, 0)
        phase(2 * p + 1, 1)
        return carry

    lax.fori_loop(0, (NCH + 1) // 2, pair, 0)
    scat_drain((NCH - 1) % 2)
    plsc.subcore_barrier()

    def out_body(j, carry):
        @pl.when(j % NS == sid)
        def _():
            pltpu.sync_copy(acc_sh.at[pl.ds(j * ZB, ZB)],
                            acc_out.at[cid, pl.ds(j * ZB, ZB)])
            ab = abuf0.at[pl.ds(0, ZB)]
            pltpu.sync_copy(asum_sh.at[pl.ds(j * ZB, ZB)], ab)
            pltpu.sync_copy(ab,
                            asum_out.at[pl.ds(cid * N + j * ZB, ZB)])
        return carry

    lax.fori_loop(0, NZC, out_body, 0)


@jax.jit
def _sc_edges(feat, el, er, src, dst, cvec):
    mesh = plsc.VectorSubcoreMesh(core_axis_name="c", subcore_axis_name="s")
    return pl.kernel(
        _sc_body,
        out_type=(jax.ShapeDtypeStruct((NC, N, D), jnp.float32),
                  jax.ShapeDtypeStruct((NC * N,), jnp.float32)),
        mesh=mesh,
        scratch_types=[
            pltpu.VMEM((N,), jnp.float32),
            pltpu.VMEM((N,), jnp.float32),
            pltpu.VMEM((L,), jnp.float32),
            pltpu.VMEM((K,), jnp.int32),
            pltpu.VMEM((K,), jnp.int32),
            pltpu.VMEM((K,), jnp.int32),
            pltpu.VMEM((K,), jnp.int32),
            pltpu.VMEM((K,), jnp.int32),
            pltpu.VMEM((K,), jnp.int32),
            pltpu.VMEM((K, D), jnp.float32),
            pltpu.VMEM((K, D), jnp.float32),
            pltpu.VMEM((K,), jnp.float32),
            pltpu.VMEM((K,), jnp.float32),
            pltpu.VMEM((ZB, D), jnp.float32),
            pltpu.VMEM_SHARED((N, D), jnp.float32),
            pltpu.VMEM_SHARED((N,), jnp.float32),
            pltpu.SemaphoreType.DMA,
            pltpu.SemaphoreType.DMA,
            pltpu.SemaphoreType.DMA,
            pltpu.SemaphoreType.DMA,
            pltpu.SemaphoreType.DMA,
            pltpu.SemaphoreType.DMA,
            pltpu.SemaphoreType.DMA,
            pltpu.SemaphoreType.DMA,
        ],
        compiler_params=pltpu.CompilerParams(needs_layout_passes=False),
    )(feat, el, er, src, dst, cvec)


def _finish_body(acc_ref, asum_ref, x_ref, b_ref, o_ref):
    s = acc_ref[0] + acc_ref[1]
    denom = asum_ref[0] + asum_ref[1] + jnp.float32(1e-9)
    o_ref[...] = s / denom + x_ref[...] + b_ref[...]


@jax.jit
def _finish(acc, asum, x, bias2):
    return pl.pallas_call(
        _finish_body,
        grid=(N // BR,),
        in_specs=[
            pl.BlockSpec((NC, BR, D), lambda i: (0, i, 0)),
            pl.BlockSpec((NC, BR, 1), lambda i: (0, i, 0)),
            pl.BlockSpec((BR, D), lambda i: (i, 0)),
            pl.BlockSpec((1, D), lambda i: (0, 0)),
        ],
        out_specs=pl.BlockSpec((BR, D), lambda i: (i, 0)),
        out_shape=jax.ShapeDtypeStruct((N, D), jnp.float32),
    )(acc, asum, x, bias2)


def kernel(x, edge_index, W, attn_l, attn_r, bias):
    src = edge_index[0].astype(jnp.int32)
    dst = edge_index[1].astype(jnp.int32)
    feat, el2, er2, ml, mr = _prep(x, W, attn_l.reshape(D, 1),
                                   attn_r.reshape(D, 1))
    C = jnp.maximum(ml[0, 0] + mr[0, 0], jnp.float32(0.0))
    cvec = jnp.full((L,), C, jnp.float32)
    acc, asum = _sc_edges(feat, el2.reshape(N), er2.reshape(N), src, dst,
                          cvec)
    return _finish(acc, asum.reshape(NC, N, 1), x, bias.reshape(1, D))
